# Initial kernel scaffold; baseline (speedup 1.0000x reference)
import jax
import jax.numpy as jnp
from jax.experimental import pallas as pl


def kernel(data, embeddings):
    raise NotImplementedError("write your pallas kernel here")


# speedup vs baseline: 5.0058x; 5.0058x over previous
import functools

import jax
import jax.numpy as jnp
from jax import lax
from jax.experimental import pallas as pl
from jax.experimental.pallas import tpu as pltpu
from jax.experimental.pallas import tpu_sc as plsc

_R, _S = 16384, 26
_V, _D = 10000, 64
_N = _R * _S
_NC, _NS = 2, 16
_NW = _NC * _NS
_PER_W = _N // _NW
_CHUNK = 512
_N_CHUNKS = _PER_W // _CHUNK


@functools.partial(jax.jit, static_argnames=())
def _sc_gather(idx_flat, table):
    mesh = plsc.VectorSubcoreMesh(core_axis_name="c", subcore_axis_name="s")

    @functools.partial(
        pl.kernel,
        mesh=mesh,
        out_type=jax.ShapeDtypeStruct((_N, _D), jnp.float32),
        scratch_types=[
            pltpu.VMEM((_CHUNK,), jnp.int32),
            pltpu.VMEM((_CHUNK, _D), jnp.float32),
            pltpu.SemaphoreType.DMA,
        ],
        compiler_params=pltpu.CompilerParams(use_tc_tiling_on_sc=False),
    )
    def k(idx_hbm, table_hbm, out_hbm, idx_v, rows_v, sem):
        wid = lax.axis_index("s") * _NC + lax.axis_index("c")
        base = wid * _PER_W

        def body(i, carry):
            off = base + i * _CHUNK
            pltpu.sync_copy(idx_hbm.at[pl.ds(off, _CHUNK)], idx_v)
            pltpu.async_copy(table_hbm.at[idx_v], rows_v, sem).wait()
            pltpu.sync_copy(rows_v, out_hbm.at[pl.ds(off, _CHUNK)])
            return carry

        lax.fori_loop(0, _N_CHUNKS, body, 0)

    return k(idx_flat, table)


def kernel(data, embeddings):
    idx = data.reshape(-1).astype(jnp.int32)
    out = _sc_gather(idx, embeddings)
    return out.reshape(data.shape + (embeddings.shape[1],))


# speedup vs baseline: 5.2733x; 1.0534x over previous
import functools

import jax
import jax.numpy as jnp
from jax import lax
from jax.experimental import pallas as pl
from jax.experimental.pallas import tpu as pltpu
from jax.experimental.pallas import tpu_sc as plsc

_R, _S = 16384, 26
_V, _D = 10000, 64
_N = _R * _S
_NC, _NS = 2, 16
_NW = _NC * _NS
_PER_W = _N // _NW
_CHUNK = 512
_N_CHUNKS = _PER_W // _CHUNK
_N_PAIRS = _N_CHUNKS // 2


def _sc_gather(idx3, table):
    mesh = plsc.VectorSubcoreMesh(core_axis_name="c", subcore_axis_name="s")

    @functools.partial(
        pl.kernel,
        mesh=mesh,
        out_type=jax.ShapeDtypeStruct((_N, _D), jnp.float32),
        scratch_types=[
            pltpu.VMEM((_N_CHUNKS, _CHUNK), jnp.int32),
            pltpu.VMEM((_CHUNK, _D), jnp.float32),
            pltpu.VMEM((_CHUNK, _D), jnp.float32),
            pltpu.SemaphoreType.DMA,
            pltpu.SemaphoreType.DMA,
            pltpu.SemaphoreType.DMA,
            pltpu.SemaphoreType.DMA,
        ],
        compiler_params=pltpu.CompilerParams(use_tc_tiling_on_sc=False),
    )
    def k(idx_hbm, table_hbm, out_hbm, idx_all, rows0, rows1, sg0, sg1, sw0, sw1):
        wid = lax.axis_index("s") * _NC + lax.axis_index("c")
        base = wid * _PER_W
        pltpu.sync_copy(idx_hbm.at[wid], idx_all)

        def gather(i, buf, sem):
            pltpu.async_copy(table_hbm.at[idx_all.at[i]], buf, sem)

        def wait_gather(i, buf, sem):
            pltpu.make_async_copy(table_hbm.at[idx_all.at[i]], buf, sem).wait()

        def write(i, buf, sem):
            pltpu.async_copy(buf, out_hbm.at[pl.ds(base + i * _CHUNK, _CHUNK)], sem)

        def wait_write(i, buf, sem):
            pltpu.make_async_copy(
                buf, out_hbm.at[pl.ds(base + i * _CHUNK, _CHUNK)], sem
            ).wait()

        gather(0, rows0, sg0)

        def body(g, carry):
            c0 = 2 * g
            c1 = c0 + 1
            wait_gather(c0, rows0, sg0)
            write(c0, rows0, sw0)

            @pl.when(g > 0)
            def _():
                wait_write(c0 - 1, rows1, sw1)

            gather(c1, rows1, sg1)
            wait_gather(c1, rows1, sg1)
            write(c1, rows1, sw1)
            wait_write(c0, rows0, sw0)

            @pl.when(g < _N_PAIRS - 1)
            def _():
                gather(c0 + 2, rows0, sg0)

            return carry

        lax.fori_loop(0, _N_PAIRS, body, 0)
        wait_write(_N_CHUNKS - 1, rows1, sw1)

    return k(idx3, table)


def kernel(data, embeddings):
    idx = data.reshape(_NW, _N_CHUNKS, _CHUNK).astype(jnp.int32)
    out = _sc_gather(idx, embeddings)
    return out.reshape(data.shape + (embeddings.shape[1],))


# speedup vs baseline: 5.2810x; 1.0015x over previous
import functools

import jax
import jax.numpy as jnp
from jax import lax
from jax.experimental import pallas as pl
from jax.experimental.pallas import tpu as pltpu
from jax.experimental.pallas import tpu_sc as plsc

_R, _S = 16384, 26
_V, _D = 10000, 64
_N = _R * _S
_L = 128
_NC, _NS = 2, 16
_NW = _NC * _NS
_PER_W = _N // _NW
_IROWS_W = _PER_W // _L
_IROWS_C = 4
_CHUNK = _IROWS_C * _L
_N_CHUNKS = _IROWS_W // _IROWS_C
_N_PAIRS = _N_CHUNKS // 2


def _sc_gather(idx2d, table):
    mesh = plsc.VectorSubcoreMesh(core_axis_name="c", subcore_axis_name="s")

    @functools.partial(
        pl.kernel,
        mesh=mesh,
        out_type=jax.ShapeDtypeStruct((_N, _D), jnp.float32),
        scratch_types=[
            pltpu.VMEM((_IROWS_W, _L), jnp.int32),
            pltpu.VMEM((_CHUNK, _D), jnp.float32),
            pltpu.VMEM((_CHUNK, _D), jnp.float32),
            pltpu.SemaphoreType.DMA,
            pltpu.SemaphoreType.DMA,
            pltpu.SemaphoreType.DMA,
            pltpu.SemaphoreType.DMA,
        ],
        compiler_params=pltpu.CompilerParams(use_tc_tiling_on_sc=False),
    )
    def k(idx_hbm, table_hbm, out_hbm, idx_all, rows0, rows1, sg0, sg1, sw0, sw1):
        wid = lax.axis_index("s") * _NC + lax.axis_index("c")
        base = wid * _PER_W
        pltpu.sync_copy(idx_hbm.at[pl.ds(wid * _IROWS_W, _IROWS_W)], idx_all)

        def gather(c, buf, sem):
            for j in range(_IROWS_C):
                pltpu.async_copy(
                    table_hbm.at[idx_all.at[c * _IROWS_C + j]],
                    buf.at[pl.ds(j * _L, _L)],
                    sem,
                )

        def wait_gather(c, buf, sem):
            for j in range(_IROWS_C):
                pltpu.make_async_copy(
                    table_hbm.at[idx_all.at[c * _IROWS_C + j]],
                    buf.at[pl.ds(j * _L, _L)],
                    sem,
                ).wait()

        def write(c, buf, sem):
            pltpu.async_copy(buf, out_hbm.at[pl.ds(base + c * _CHUNK, _CHUNK)], sem)

        def wait_write(c, buf, sem):
            pltpu.make_async_copy(
                buf, out_hbm.at[pl.ds(base + c * _CHUNK, _CHUNK)], sem
            ).wait()

        gather(0, rows0, sg0)

        def body(g, carry):
            c0 = 2 * g
            c1 = c0 + 1
            wait_gather(c0, rows0, sg0)
            write(c0, rows0, sw0)

            @pl.when(g > 0)
            def _():
                wait_write(c0 - 1, rows1, sw1)

            gather(c1, rows1, sg1)
            wait_gather(c1, rows1, sg1)
            write(c1, rows1, sw1)
            wait_write(c0, rows0, sw0)

            @pl.when(g < _N_PAIRS - 1)
            def _():
                gather(c0 + 2, rows0, sg0)

            return carry

        lax.fori_loop(0, _N_PAIRS, body, 0)
        wait_write(_N_CHUNKS - 1, rows1, sw1)

    return k(idx2d, table)


def kernel(data, embeddings):
    idx2d = data.reshape(_N // _L, _L)
    out = _sc_gather(idx2d, embeddings)
    return out.reshape(data.shape + (embeddings.shape[1],))
